# baseline (speedup 1.0000x reference)
import jax
import jax.numpy as jnp
from jax import lax
from jax.experimental import pallas as pl
from jax.experimental.pallas import tpu as pltpu
from jax.experimental.pallas import tpu_sc as plsc

B = 16384
D = 32
H = 50
V_U = 100000
V_I = 1000000
NC = 2
NS = 16
NW = NC * NS
BW = B // NW
HC = 8
NCH = BW // HC
NG = BW // 16


BV = 512
DP = D + 1
BW1 = BV * DP


def _transpose_table(wid, wt, tail, out1d, v_total, tail_wid,
                     tbuf, obuf, sin, sout):
    nb = v_total // BV
    rem = v_total - nb * BV
    iotap = lax.iota(jnp.int32, 16) * DP
    ni = (nb - wid + NW - 1) // NW

    def transpose_rows(tb, ob, ncols):
        @pl.loop(0, D)
        def _d(d):
            for c0 in range(0, ncols, 16):
                vals = tb[d, pl.ds(c0, 16)]
                plsc.store_scatter(ob, [iotap + (c0 * DP + d)], vals)

    def in_cp(i, k):
        v0 = pl.multiple_of((wid + i * NW) * BV, 128)
        return pltpu.make_async_copy(wt.at[:, pl.ds(v0, BV)], tbuf[k], sin[k])

    def out_cp(i, k):
        v0 = pl.multiple_of((wid + i * NW) * BV, 128)
        return pltpu.make_async_copy(obuf[k], out1d.at[pl.ds(v0 * DP, BW1)],
                                     sout[k])

    for k in range(2):
        @pl.when(k < ni)
        def _():
            in_cp(k, k).start()

    @pl.loop(0, ni)
    def _blk(i):
        k = lax.rem(i, 2)
        for kk in range(2):
            @pl.when(k == kk)
            def _():
                in_cp(i, kk).wait()

                @pl.when(i >= 2)
                def _():
                    out_cp(i - 2, kk).wait()

                transpose_rows(tbuf[kk], obuf[kk], BV)
                out_cp(i, kk).start()

                @pl.when(i + 2 < ni)
                def _():
                    in_cp(i + 2, kk).start()

    for d in range(1, 3):
        @pl.when(ni >= d)
        def _():
            for kk in range(2):
                @pl.when(lax.rem(ni - d, 2) == kk)
                def _():
                    out_cp(ni - d, kk).wait()

    nstripe = rem // 128
    for st in range(nstripe):
        @pl.when(wid == tail_wid + 1 + st)
        def _():
            v0 = (nb * BV) + st * 128
            pltpu.sync_copy(wt.at[:, pl.ds(v0, 128)], tbuf[0].at[:, 0:128])
            transpose_rows(tbuf[0], obuf[0], 128)
            pltpu.sync_copy(obuf[0].at[pl.ds(0, 128 * DP)],
                            out1d.at[pl.ds(v0 * DP, 128 * DP)])

    tail_nv = rem - nstripe * 128
    if tail_nv:
        @pl.when(wid == tail_wid)
        def _():
            v0 = nb * BV + nstripe * 128
            pltpu.sync_copy(tail, obuf[1].at[pl.ds(0, tail_nv * D)])

            @pl.loop(0, tail_nv)
            def _r(r):
                obuf[0][pl.ds(r * DP, 16)] = obuf[1][pl.ds(r * D, 16)]
                obuf[0][pl.ds(r * DP + 16, 16)] = obuf[1][pl.ds(r * D + 16,
                                                                16)]

            pltpu.sync_copy(obuf[0].at[pl.ds(0, tail_nv * DP)],
                            out1d.at[pl.ds(v0 * DP, tail_nv * DP)])


def _tp_body(wtu, wti, wth, tail_u, tail_i, tail_h,
             out_u, out_i, out_h, tbuf0, tbuf1, obuf0, obuf1,
             sin0, sin1, sout0, sout1):
    wid = lax.axis_index("s") * NC + lax.axis_index("c")
    tbuf = (tbuf0, tbuf1)
    obuf = (obuf0, obuf1)
    sin = (sin0, sin1)
    sout = (sout0, sout1)
    _transpose_table(wid, wtu, tail_u, out_u, V_U, 0, tbuf, obuf, sin, sout)
    _transpose_table(wid, wti, tail_i, out_i, V_I, 1, tbuf, obuf, sin, sout)
    _transpose_table(wid, wth, tail_h, out_h, V_I, 2, tbuf, obuf, sin, sout)


def _accum_chunk(cc, hidx_k, hrows_k, out_v):
    mn = hidx_k[pl.ds(0, 16)]
    for t in range(1, HC * H // 16):
        mn = jnp.minimum(mn, hidx_k[pl.ds(t * 16, 16)])
    clean = plsc.all_reduce_population_count(mn == 0)[0] == 0

    @pl.when(clean)
    def _():
        @pl.loop(0, HC)
        def _b(b):
            bb = b * H
            acc0 = hrows_k[bb, 0:16]
            acc1 = hrows_k[bb, 16:32]
            for j in range(1, H):
                acc0 += hrows_k[bb + j, 0:16]
                acc1 += hrows_k[bb + j, 16:32]
            row = cc * HC + b
            out_v[row, 32:48] = acc0
            out_v[row, 48:64] = acc1

    @pl.when(jnp.logical_not(clean))
    def _():
        @pl.loop(0, HC)
        def _b(b):
            bb = b * H
            v0 = hidx_k[pl.ds(bb, 16)]
            v1 = hidx_k[pl.ds(bb + 16, 16)]
            v2 = hidx_k[pl.ds(bb + 32, 16)]
            v3 = hidx_k[pl.ds(bb + 34, 16)]
            m0 = jnp.where(v0 == 0, 0.0, 1.0)
            m1 = jnp.where(v1 == 0, 0.0, 1.0)
            m2 = jnp.where(v2 == 0, 0.0, 1.0)
            m3 = jnp.where(v3 == 0, 0.0, 1.0)
            acc0 = jnp.zeros((16,), jnp.float32)
            acc1 = jnp.zeros((16,), jnp.float32)
            for j in range(H):
                if j < 16:
                    m = m0[j]
                elif j < 32:
                    m = m1[j - 16]
                elif j < 48:
                    m = m2[j - 32]
                else:
                    m = m3[j - 34]
                acc0 += hrows_k[bb + j, 0:16] * m
                acc1 += hrows_k[bb + j, 16:32] * m
            row = cc * HC + b
            out_v[row, 32:48] = acc0
            out_v[row, 48:64] = acc1


def _emb_body(uid_hbm, iid_hbm, hidx_hbm, wu_hbm, wi_hbm, wh_hbm,
              user_out, item_out,
              uid_v, iid_v, hidx0, hidx1, urows, irows, hrows0, hrows1,
              out_v, sem_u, sem_i, sem_h0, sem_h1):
    wid = lax.axis_index("s") * NC + lax.axis_index("c")
    base = wid * BW
    hidx = (hidx0, hidx1)
    hrows = (hrows0, hrows1)
    sem_h = (sem_h0, sem_h1)

    pltpu.sync_copy(uid_hbm.at[pl.ds(base, BW)], uid_v)
    cp_u = pltpu.async_copy(wu_hbm.at[uid_v], urows, sem_u)
    pltpu.sync_copy(iid_hbm.at[pl.ds(base, BW)], iid_v)
    cp_i = pltpu.async_copy(wi_hbm.at[iid_v], irows, sem_i)

    for k in range(2):
        hbase = (base + k * HC) * H
        pltpu.sync_copy(hidx_hbm.at[pl.ds(hbase, HC * H)], hidx[k])
        pltpu.async_copy(wh_hbm.at[hidx[k]], hrows[k], sem_h[k])

    @pl.loop(0, NCH, step=2)
    def _hist(c):
        for k in range(2):
            cc = c + k
            pltpu.make_async_copy(wh_hbm.at[hidx[k]], hrows[k],
                                  sem_h[k]).wait()
            _accum_chunk(cc, hidx[k], hrows[k], out_v)

            @pl.when(cc + 2 < NCH)
            def _():
                hbase2 = (base + (cc + 2) * HC) * H
                pltpu.sync_copy(hidx_hbm.at[pl.ds(hbase2, HC * H)], hidx[k])
                pltpu.async_copy(wh_hbm.at[hidx[k]], hrows[k], sem_h[k])

    cp_i.wait()

    @pl.loop(0, NG)
    def _item(g):
        r0 = g * 16
        v = iid_v[pl.ds(r0, 16)]
        clean = plsc.all_reduce_population_count(v == 0)[0] == 0

        @pl.when(clean)
        def _():
            for l in range(16):
                out_v[r0 + l, 0:16] = irows[r0 + l, 0:16]
                out_v[r0 + l, 16:32] = irows[r0 + l, 16:32]

        @pl.when(jnp.logical_not(clean))
        def _():
            for l in range(16):
                m = jnp.where(v[l] == 0, 0.0, 1.0)
                out_v[r0 + l, 0:16] = irows[r0 + l, 0:16] * m
                out_v[r0 + l, 16:32] = irows[r0 + l, 16:32] * m

    pltpu.sync_copy(out_v, item_out.at[pl.ds(base, BW)])

    cp_u.wait()

    @pl.loop(0, NG)
    def _user(g):
        r0 = g * 16
        v = uid_v[pl.ds(r0, 16)]

        @pl.when(plsc.all_reduce_population_count(v == 0)[0] != 0)
        def _():
            for l in range(16):
                m = jnp.where(v[l] == 0, 0.0, 1.0)
                urows[r0 + l, 0:16] = urows[r0 + l, 0:16] * m
                urows[r0 + l, 16:32] = urows[r0 + l, 16:32] * m

    pltpu.sync_copy(urows.at[:, 0:D], user_out.at[pl.ds(base, BW)])


@jax.jit
def kernel(user_id, item_id, item_hist, W_user_id, W_item_id, W_item_hist):
    hist_flat = item_hist.reshape(B * H)
    mesh = plsc.VectorSubcoreMesh(core_axis_name="c", subcore_axis_name="s")

    tp = pl.kernel(
        _tp_body,
        out_type=(jax.ShapeDtypeStruct((V_U * DP,), jnp.float32),
                  jax.ShapeDtypeStruct((V_I * DP,), jnp.float32),
                  jax.ShapeDtypeStruct((V_I * DP,), jnp.float32)),
        mesh=mesh,
        compiler_params=pltpu.CompilerParams(needs_layout_passes=False,
                                             use_tc_tiling_on_sc=True),
        scratch_types=[
            pltpu.VMEM((D, BV), jnp.float32),
            pltpu.VMEM((D, BV), jnp.float32),
            pltpu.VMEM((BW1,), jnp.float32),
            pltpu.VMEM((BW1,), jnp.float32),
            pltpu.SemaphoreType.DMA,
            pltpu.SemaphoreType.DMA,
            pltpu.SemaphoreType.DMA,
            pltpu.SemaphoreType.DMA,
        ],
    )
    wu1d, wi1d, wh1d = tp(W_user_id.T, W_item_id.T, W_item_hist.T,
                          W_user_id[(V_U // 128) * 128:].reshape(-1),
                          W_item_id[(V_I // 128) * 128:].reshape(-1),
                          W_item_hist[(V_I // 128) * 128:].reshape(-1))

    f = pl.kernel(
        _emb_body,
        out_type=(jax.ShapeDtypeStruct((B, D), jnp.float32),
                  jax.ShapeDtypeStruct((B, 2 * D), jnp.float32)),
        mesh=mesh,
        compiler_params=pltpu.CompilerParams(needs_layout_passes=False,
                                             use_tc_tiling_on_sc=False),
        scratch_types=[
            pltpu.VMEM((BW,), jnp.int32),
            pltpu.VMEM((BW,), jnp.int32),
            pltpu.VMEM((HC * H,), jnp.int32),
            pltpu.VMEM((HC * H,), jnp.int32),
            pltpu.VMEM((BW, DP), jnp.float32),
            pltpu.VMEM((BW, DP), jnp.float32),
            pltpu.VMEM((HC * H, DP), jnp.float32),
            pltpu.VMEM((HC * H, DP), jnp.float32),
            pltpu.VMEM((BW, 2 * D), jnp.float32),
            pltpu.SemaphoreType.DMA,
            pltpu.SemaphoreType.DMA,
            pltpu.SemaphoreType.DMA,
            pltpu.SemaphoreType.DMA,
        ],
    )
    user_emb, item_out = f(user_id, item_id, hist_flat,
                           wu1d.reshape(V_U, DP), wi1d.reshape(V_I, DP),
                           wh1d.reshape(V_I, DP))
    return (user_emb, item_out)


# speedup vs baseline: 3.5380x; 3.5380x over previous
import jax
import jax.numpy as jnp
from jax import lax
from jax.experimental import pallas as pl
from jax.experimental.pallas import tpu as pltpu
from jax.experimental.pallas import tpu_sc as plsc

B = 16384
D = 32
H = 50
NC = 2
NS = 16
NW = NC * NS
BW = B // NW
HC = 16
NCH = BW // HC
NG = BW // 16


def _accum_chunk(cc, hidx_k, hrows_k, out_v):
    mn = hidx_k[pl.ds(0, 16)]
    for t in range(1, HC * H // 16):
        mn = jnp.minimum(mn, hidx_k[pl.ds(t * 16, 16)])
    clean = plsc.all_reduce_population_count(mn == 0)[0] == 0

    @pl.when(clean)
    def _():
        @pl.loop(0, HC)
        def _b(b):
            bb = b * H
            acc0 = hrows_k[bb, 0:16]
            acc1 = hrows_k[bb, 16:32]
            for j in range(1, H):
                acc0 += hrows_k[bb + j, 0:16]
                acc1 += hrows_k[bb + j, 16:32]
            row = cc * HC + b
            out_v[row, 32:48] = acc0
            out_v[row, 48:64] = acc1

    @pl.when(jnp.logical_not(clean))
    def _():
        @pl.loop(0, HC)
        def _b(b):
            bb = b * H
            v0 = hidx_k[pl.ds(bb, 16)]
            v1 = hidx_k[pl.ds(bb + 16, 16)]
            v2 = hidx_k[pl.ds(bb + 32, 16)]
            v3 = hidx_k[pl.ds(bb + 34, 16)]
            m0 = jnp.where(v0 == 0, 0.0, 1.0)
            m1 = jnp.where(v1 == 0, 0.0, 1.0)
            m2 = jnp.where(v2 == 0, 0.0, 1.0)
            m3 = jnp.where(v3 == 0, 0.0, 1.0)
            acc0 = jnp.zeros((16,), jnp.float32)
            acc1 = jnp.zeros((16,), jnp.float32)
            for j in range(H):
                if j < 16:
                    m = m0[j]
                elif j < 32:
                    m = m1[j - 16]
                elif j < 48:
                    m = m2[j - 32]
                else:
                    m = m3[j - 34]
                acc0 += hrows_k[bb + j, 0:16] * m
                acc1 += hrows_k[bb + j, 16:32] * m
            row = cc * HC + b
            out_v[row, 32:48] = acc0
            out_v[row, 48:64] = acc1


def _emb_body(uid_hbm, iid_hbm, hidx_hbm, wu_hbm, wi_hbm, wh_hbm,
              user_out, item_out,
              uid_v, iid_v, hidx0, hidx1, urows, irows, hrows0, hrows1,
              out_v, sem_u, sem_i, sem_h0, sem_h1):
    wid = lax.axis_index("s") * NC + lax.axis_index("c")
    base = wid * BW
    hidx = (hidx0, hidx1)
    hrows = (hrows0, hrows1)
    sem_h = (sem_h0, sem_h1)

    pltpu.sync_copy(uid_hbm.at[pl.ds(base, BW)], uid_v)
    cp_u = pltpu.async_copy(wu_hbm.at[uid_v], urows, sem_u)
    pltpu.sync_copy(iid_hbm.at[pl.ds(base, BW)], iid_v)
    cp_i = pltpu.async_copy(wi_hbm.at[iid_v], irows, sem_i)

    for k in range(2):
        hbase = (base + k * HC) * H
        pltpu.sync_copy(hidx_hbm.at[pl.ds(hbase, HC * H)], hidx[k])
        pltpu.async_copy(wh_hbm.at[hidx[k]], hrows[k], sem_h[k])

    @pl.loop(0, NCH, step=2)
    def _hist(c):
        for k in range(2):
            cc = c + k
            pltpu.make_async_copy(wh_hbm.at[hidx[k]], hrows[k],
                                  sem_h[k]).wait()
            _accum_chunk(cc, hidx[k], hrows[k], out_v)

            @pl.when(cc + 2 < NCH)
            def _():
                hbase2 = (base + (cc + 2) * HC) * H
                pltpu.sync_copy(hidx_hbm.at[pl.ds(hbase2, HC * H)], hidx[k])
                pltpu.async_copy(wh_hbm.at[hidx[k]], hrows[k], sem_h[k])

    cp_i.wait()

    @pl.loop(0, NG)
    def _item(g):
        r0 = g * 16
        v = iid_v[pl.ds(r0, 16)]
        clean = plsc.all_reduce_population_count(v == 0)[0] == 0

        @pl.when(clean)
        def _():
            for l in range(16):
                out_v[r0 + l, 0:16] = irows[r0 + l, 0:16]
                out_v[r0 + l, 16:32] = irows[r0 + l, 16:32]

        @pl.when(jnp.logical_not(clean))
        def _():
            for l in range(16):
                m = jnp.where(v[l] == 0, 0.0, 1.0)
                out_v[r0 + l, 0:16] = irows[r0 + l, 0:16] * m
                out_v[r0 + l, 16:32] = irows[r0 + l, 16:32] * m

    pltpu.sync_copy(out_v, item_out.at[pl.ds(base, BW)])

    cp_u.wait()

    @pl.loop(0, NG)
    def _user(g):
        r0 = g * 16
        v = uid_v[pl.ds(r0, 16)]

        @pl.when(plsc.all_reduce_population_count(v == 0)[0] != 0)
        def _():
            for l in range(16):
                m = jnp.where(v[l] == 0, 0.0, 1.0)
                urows[r0 + l, 0:16] = urows[r0 + l, 0:16] * m
                urows[r0 + l, 16:32] = urows[r0 + l, 16:32] * m

    pltpu.sync_copy(urows, user_out.at[pl.ds(base, BW)])


@jax.jit
def kernel(user_id, item_id, item_hist, W_user_id, W_item_id, W_item_hist):
    hist_flat = item_hist.reshape(B * H)
    mesh = plsc.VectorSubcoreMesh(core_axis_name="c", subcore_axis_name="s")
    f = pl.kernel(
        _emb_body,
        out_type=(jax.ShapeDtypeStruct((B, D), jnp.float32),
                  jax.ShapeDtypeStruct((B, 2 * D), jnp.float32)),
        mesh=mesh,
        compiler_params=pltpu.CompilerParams(needs_layout_passes=False,
                                             use_tc_tiling_on_sc=False),
        scratch_types=[
            pltpu.VMEM((BW,), jnp.int32),
            pltpu.VMEM((BW,), jnp.int32),
            pltpu.VMEM((HC * H,), jnp.int32),
            pltpu.VMEM((HC * H,), jnp.int32),
            pltpu.VMEM((BW, D), jnp.float32),
            pltpu.VMEM((BW, D), jnp.float32),
            pltpu.VMEM((HC * H, D), jnp.float32),
            pltpu.VMEM((HC * H, D), jnp.float32),
            pltpu.VMEM((BW, 2 * D), jnp.float32),
            pltpu.SemaphoreType.DMA,
            pltpu.SemaphoreType.DMA,
            pltpu.SemaphoreType.DMA,
            pltpu.SemaphoreType.DMA,
        ],
    )
    user_emb, item_out = f(user_id, item_id, hist_flat,
                           W_user_id, W_item_id, W_item_hist)
    return (user_emb, item_out)
